# baseline (speedup 1.0000x reference)
import jax
import jax.numpy as jnp
from jax.experimental import pallas as pl
from jax.experimental.pallas import tpu as pltpu

_BLK = 512


def _proj_kernel(x_ref, wt_ref, b_ref, h_ref):
    h_ref[...] = (
        jnp.dot(x_ref[...], wt_ref[...], preferred_element_type=jnp.float32)
        + b_ref[...]
    )


def _agg_kernel(h_ref, adj_ref, out_ref):
    adj = adj_ref[...]
    acc = jnp.dot(adj, h_ref[...], preferred_element_type=jnp.float32)
    denom = jnp.sum(adj, axis=-1, keepdims=True)
    out_ref[...] = acc / denom


def kernel(node_feats, adj_matrix, W, b):
    B, N, c_in = node_feats.shape
    c_out = W.shape[0]
    x = node_feats.reshape(N, c_in)
    adj = adj_matrix.reshape(N, N)
    wt = W.T
    b2 = b.reshape(1, c_out)

    h = pl.pallas_call(
        _proj_kernel,
        out_shape=jax.ShapeDtypeStruct((N, c_out), jnp.float32),
    )(x, wt, b2)

    out = pl.pallas_call(
        _agg_kernel,
        grid=(N // _BLK,),
        in_specs=[
            pl.BlockSpec((N, c_out), lambda i: (0, 0)),
            pl.BlockSpec((_BLK, N), lambda i: (i, 0)),
        ],
        out_specs=pl.BlockSpec((_BLK, c_out), lambda i: (i, 0)),
        out_shape=jax.ShapeDtypeStruct((N, c_out), jnp.float32),
        compiler_params=pltpu.CompilerParams(
            dimension_semantics=("parallel",),
        ),
    )(h, adj)
    return out.reshape(B, N, c_out)


# speedup vs baseline: 1.1463x; 1.1463x over previous
import jax
import jax.numpy as jnp
from jax.experimental import pallas as pl
from jax.experimental.pallas import tpu as pltpu

_BLK = 512


def _gcn_block_kernel(x_ref, wt_ref, b_ref, adj_ref, out_ref, h_ref):
    i = pl.program_id(0)

    @pl.when(i == 0)
    def _project():
        h_ref[...] = (
            jnp.dot(x_ref[...], wt_ref[...], preferred_element_type=jnp.float32)
            + b_ref[...]
        )

    adj = adj_ref[...]
    acc = jnp.dot(adj, h_ref[...], preferred_element_type=jnp.float32)
    denom = jnp.sum(adj, axis=-1, keepdims=True)
    out_ref[...] = acc / denom


def kernel(node_feats, adj_matrix, W, b):
    B, N, c_in = node_feats.shape
    c_out = W.shape[0]
    x = node_feats.reshape(N, c_in)
    adj = adj_matrix.reshape(N, N)
    wt = W.T
    b2 = b.reshape(1, c_out)

    grid = (N // _BLK,)
    out = pl.pallas_call(
        _gcn_block_kernel,
        grid=grid,
        in_specs=[
            pl.BlockSpec((N, c_in), lambda i: (0, 0)),
            pl.BlockSpec((c_in, c_out), lambda i: (0, 0)),
            pl.BlockSpec((1, c_out), lambda i: (0, 0)),
            pl.BlockSpec((_BLK, N), lambda i: (i, 0)),
        ],
        out_specs=pl.BlockSpec((_BLK, c_out), lambda i: (i, 0)),
        out_shape=jax.ShapeDtypeStruct((N, c_out), jnp.float32),
        scratch_shapes=[pltpu.VMEM((N, c_out), jnp.float32)],
        compiler_params=pltpu.CompilerParams(
            dimension_semantics=("arbitrary",),
        ),
    )(x, wt, b2, adj)
    return out.reshape(B, N, c_out)
